# baseline (speedup 1.0000x reference)
import jax
import jax.numpy as jnp
import numpy as np
from jax import lax
from jax.experimental import pallas as pl
from jax.experimental.pallas import tpu as pltpu
from jax.experimental.pallas import tpu_sc as plsc

N_NODES = 100000
DIM = 16
N_TICKS = 51
BATCH = 16384
NC = 2
NS = 16
NW = NC * NS
BPW = BATCH // NW
NG = BPW // 16
NROW = BPW * 4 * DIM // 128
TSTRIDE = DIM * N_NODES
STEP = np.float32(1.0 / (N_TICKS - 1))


def _body(flat, bias16, src2, dst2, t2, out,
          src_v, dst_v, t_v, ti_v, dt_v, idx_v, val_v, out_v, bias_v, sem):
    wid = lax.axis_index("s") * NC + lax.axis_index("c")
    base = pl.multiple_of(wid * BPW, BPW)
    pltpu.sync_copy(bias16, bias_v)
    pltpu.sync_copy(src2.at[wid], src_v)
    pltpu.sync_copy(dst2.at[wid], dst_v)
    pltpu.sync_copy(t2.at[wid], t_v)

    def precomp(i, carry):
        tv = t_v[pl.ds(i * 16, 16)]
        ti_v[pl.ds(i * 16, 16)] = jnp.minimum((tv / STEP).astype(jnp.int32),
                                              N_TICKS - 2)
        dt_v[pl.ds(i * 16, 16)] = lax.rem(tv, STEP) / STEP
        return carry

    lax.fori_loop(0, NG, precomp, 0)

    def build_issue(g):
        tiv = ti_v[pl.ds(g * 16, 16)]
        sg = src_v[pl.ds(g * 16, 16)]
        dg = dst_v[pl.ds(g * 16, 16)]
        tb0 = tiv * TSTRIDE
        tb1 = tb0 + TSTRIDE

        def bd(d, carry):
            row = g * 8 + (d >> 1)
            col = (d & 1) * 64
            off = d * N_NODES
            bs = sg + off
            bdd = dg + off
            idx_v[row, pl.ds(col, 16)] = tb0 + bs
            idx_v[row, pl.ds(col + 16, 16)] = tb1 + bs
            idx_v[row, pl.ds(col + 32, 16)] = tb0 + bdd
            idx_v[row, pl.ds(col + 48, 16)] = tb1 + bdd
            return carry

        lax.fori_loop(0, DIM, bd, 0)
        for r in range(8):
            j = g * 8 + r
            pltpu.async_copy(flat.at[idx_v.at[j]], val_v.at[j], sem)

    def drain_compute(g):
        for r in range(8):
            j = g * 8 + r
            pltpu.make_async_copy(flat.at[idx_v.at[j]], val_v.at[j],
                                  sem).wait()
        dtg = dt_v[pl.ds(g * 16, 16)]

        def cd(d, acc):
            row = g * 8 + (d >> 1)
            col = (d & 1) * 64
            scur = val_v[row, pl.ds(col, 16)]
            snxt = val_v[row, pl.ds(col + 16, 16)]
            dcur = val_v[row, pl.ds(col + 32, 16)]
            dnxt = val_v[row, pl.ds(col + 48, 16)]
            dc = scur - dcur
            dn = snxt - dnxt
            diff = dc + dtg * (dn - dc)
            return acc + diff * diff

        acc = lax.fori_loop(0, DIM, cd, jnp.zeros((16,), jnp.float32))
        out_v[pl.ds(g * 16, 16)] = bias_v[...] - acc

    build_issue(0)
    build_issue(1)

    def step(i, carry):
        build_issue(i + 2)
        drain_compute(i)
        return carry

    lax.fori_loop(0, NG - 2, step, 0)
    drain_compute(NG - 2)
    drain_compute(NG - 1)
    pltpu.sync_copy(out_v, out.at[pl.ds(base, BPW)])


def kernel(z, bias, src, dst, t):
    flat = jnp.transpose(z, (2, 1, 0)).reshape(-1)
    bias16 = jnp.broadcast_to(jnp.asarray(bias, jnp.float32), (16,))
    src2 = src.astype(jnp.int32).reshape(NW, BPW)
    dst2 = dst.astype(jnp.int32).reshape(NW, BPW)
    t2 = t.reshape(NW, BPW)
    fn = pl.kernel(
        _body,
        out_type=jax.ShapeDtypeStruct((BATCH,), jnp.float32),
        mesh=plsc.VectorSubcoreMesh(core_axis_name="c", subcore_axis_name="s"),
        compiler_params=pltpu.CompilerParams(use_tc_tiling_on_sc=False,
                                             needs_layout_passes=False),
        scratch_types=[
            pltpu.VMEM((BPW,), jnp.int32),
            pltpu.VMEM((BPW,), jnp.int32),
            pltpu.VMEM((BPW,), jnp.float32),
            pltpu.VMEM((BPW,), jnp.int32),
            pltpu.VMEM((BPW,), jnp.float32),
            pltpu.VMEM((NROW, 128), jnp.int32),
            pltpu.VMEM((NROW, 128), jnp.float32),
            pltpu.VMEM((BPW,), jnp.float32),
            pltpu.VMEM((16,), jnp.float32),
            pltpu.SemaphoreType.DMA,
        ],
    )
    return fn(flat, bias16, src2, dst2, t2)


# speedup vs baseline: 1.0012x; 1.0012x over previous
import jax
import jax.numpy as jnp
import numpy as np
from jax import lax
from jax.experimental import pallas as pl
from jax.experimental.pallas import tpu as pltpu
from jax.experimental.pallas import tpu_sc as plsc

N_NODES = 100000
DIM = 16
N_TICKS = 51
BATCH = 16384
NC = 2
NS = 16
NW = NC * NS
BPW = BATCH // NW
NG = BPW // 16
NROW = BPW * 4 * DIM // 128
TSTRIDE = DIM * N_NODES
STEP = np.float32(1.0 / (N_TICKS - 1))


def _body(flat, bias16, src2, dst2, t2, out,
          src_v, dst_v, t_v, ti_v, dt_v, idx_v, val_v, out_v, bias_v, sem):
    wid = lax.axis_index("s") * NC + lax.axis_index("c")
    base = pl.multiple_of(wid * BPW, BPW)
    pltpu.sync_copy(bias16, bias_v)
    pltpu.sync_copy(src2.at[wid], src_v)
    pltpu.sync_copy(dst2.at[wid], dst_v)
    pltpu.sync_copy(t2.at[wid], t_v)

    def precomp(i, carry):
        tv = t_v[pl.ds(i * 16, 16)]
        ti_v[pl.ds(i * 16, 16)] = jnp.minimum((tv / STEP).astype(jnp.int32),
                                              N_TICKS - 2)
        dt_v[pl.ds(i * 16, 16)] = lax.rem(tv, STEP) / STEP
        return carry

    lax.fori_loop(0, NG, precomp, 0)

    def build_issue(g):
        tiv = ti_v[pl.ds(g * 16, 16)]
        sg = src_v[pl.ds(g * 16, 16)]
        dg = dst_v[pl.ds(g * 16, 16)]
        tb0 = tiv * TSTRIDE
        tb1 = tb0 + TSTRIDE

        def bd(d, carry):
            row = g * 8 + (d >> 1)
            col = (d & 1) * 64
            off = d * N_NODES
            bs = sg + off
            bdd = dg + off
            idx_v[row, pl.ds(col, 16)] = tb0 + bs
            idx_v[row, pl.ds(col + 16, 16)] = tb1 + bs
            idx_v[row, pl.ds(col + 32, 16)] = tb0 + bdd
            idx_v[row, pl.ds(col + 48, 16)] = tb1 + bdd
            return carry

        lax.fori_loop(0, DIM, bd, 0)
        for r in range(8):
            j = g * 8 + r
            pltpu.async_copy(flat.at[idx_v.at[j]], val_v.at[j], sem)

    def drain_compute(g):
        for r in range(8):
            j = g * 8 + r
            pltpu.make_async_copy(flat.at[idx_v.at[j]], val_v.at[j],
                                  sem).wait()
        dtg = dt_v[pl.ds(g * 16, 16)]

        def cd(d, acc):
            row = g * 8 + (d >> 1)
            col = (d & 1) * 64
            scur = val_v[row, pl.ds(col, 16)]
            snxt = val_v[row, pl.ds(col + 16, 16)]
            dcur = val_v[row, pl.ds(col + 32, 16)]
            dnxt = val_v[row, pl.ds(col + 48, 16)]
            dc = scur - dcur
            dn = snxt - dnxt
            diff = dc + dtg * (dn - dc)
            return acc + diff * diff

        acc = lax.fori_loop(0, DIM, cd, jnp.zeros((16,), jnp.float32))
        out_v[pl.ds(g * 16, 16)] = bias_v[...] - acc

    build_issue(0)
    build_issue(1)

    def step(i, carry):
        build_issue(i + 2)
        drain_compute(i)
        return carry

    lax.fori_loop(0, NG - 2, step, 0)
    drain_compute(NG - 2)
    drain_compute(NG - 1)
    pltpu.sync_copy(out_v, out.at[pl.ds(base, BPW)])


def kernel(z, bias, src, dst, t):
    flat = jnp.transpose(z, (2, 1, 0)).reshape(-1)
    bias16 = jnp.broadcast_to(jnp.asarray(bias, jnp.float32), (16,))
    src2 = src.astype(jnp.int32).reshape(NW, BPW)
    dst2 = dst.astype(jnp.int32).reshape(NW, BPW)
    t2 = t.reshape(NW, BPW)
    fn = pl.kernel(
        _body,
        out_type=jax.ShapeDtypeStruct((BATCH,), jnp.float32),
        mesh=plsc.VectorSubcoreMesh(core_axis_name="c", subcore_axis_name="s"),
        compiler_params=pltpu.CompilerParams(use_tc_tiling_on_sc=False,
                                             needs_layout_passes=False,
                                             skip_device_barrier=True),
        scratch_types=[
            pltpu.VMEM((BPW,), jnp.int32),
            pltpu.VMEM((BPW,), jnp.int32),
            pltpu.VMEM((BPW,), jnp.float32),
            pltpu.VMEM((BPW,), jnp.int32),
            pltpu.VMEM((BPW,), jnp.float32),
            pltpu.VMEM((NROW, 128), jnp.int32),
            pltpu.VMEM((NROW, 128), jnp.float32),
            pltpu.VMEM((BPW,), jnp.float32),
            pltpu.VMEM((16,), jnp.float32),
            pltpu.SemaphoreType.DMA,
        ],
    )
    return fn(flat, bias16, src2, dst2, t2)


# speedup vs baseline: 1.0040x; 1.0028x over previous
import jax
import jax.numpy as jnp
import numpy as np
from jax import lax
from jax.experimental import pallas as pl
from jax.experimental.pallas import tpu as pltpu
from jax.experimental.pallas import tpu_sc as plsc

N_NODES = 100000
DIM = 16
N_TICKS = 51
BATCH = 16384
NC = 2
NS = 16
NW = NC * NS
BPW = BATCH // NW
NG = BPW // 16
NROW = 32
TSTRIDE = DIM * N_NODES
STEP = np.float32(1.0 / (N_TICKS - 1))


def _body(flat, bias16, src2, dst2, t2, out,
          src_v, dst_v, t_v, ti_v, dt_v, idx_v, val_v, out_v, bias_v, sem):
    wid = lax.axis_index("s") * NC + lax.axis_index("c")
    base = pl.multiple_of(wid * BPW, BPW)
    pltpu.sync_copy(bias16, bias_v)
    pltpu.sync_copy(src2.at[wid], src_v)
    pltpu.sync_copy(dst2.at[wid], dst_v)
    pltpu.sync_copy(t2.at[wid], t_v)

    def precomp(i, carry):
        tv = t_v[pl.ds(i * 16, 16)]
        ti_v[pl.ds(i * 16, 16)] = jnp.minimum((tv / STEP).astype(jnp.int32),
                                              N_TICKS - 2)
        dt_v[pl.ds(i * 16, 16)] = lax.rem(tv, STEP) / STEP
        return carry

    lax.fori_loop(0, NG, precomp, 0)

    def build_issue(g):
        tiv = ti_v[pl.ds(g * 16, 16)]
        sg = src_v[pl.ds(g * 16, 16)]
        dg = dst_v[pl.ds(g * 16, 16)]
        tb0 = tiv * TSTRIDE
        tb1 = tb0 + TSTRIDE
        rb = (g & 3) * 8

        def bd(d, carry):
            row = rb + (d >> 1)
            col = (d & 1) * 64
            off = d * N_NODES
            bs = sg + off
            bdd = dg + off
            idx_v[row, pl.ds(col, 16)] = tb0 + bs
            idx_v[row, pl.ds(col + 16, 16)] = tb1 + bs
            idx_v[row, pl.ds(col + 32, 16)] = tb0 + bdd
            idx_v[row, pl.ds(col + 48, 16)] = tb1 + bdd
            return carry

        lax.fori_loop(0, DIM, bd, 0)
        for r in range(8):
            j = rb + r
            pltpu.async_copy(flat.at[idx_v.at[j]], val_v.at[j], sem)

    def drain_compute(g):
        rb = (g & 3) * 8
        for r in range(8):
            j = rb + r
            pltpu.make_async_copy(flat.at[idx_v.at[j]], val_v.at[j],
                                  sem).wait()
        dtg = dt_v[pl.ds(g * 16, 16)]

        def cd(d, acc):
            row = rb + (d >> 1)
            col = (d & 1) * 64
            scur = val_v[row, pl.ds(col, 16)]
            snxt = val_v[row, pl.ds(col + 16, 16)]
            dcur = val_v[row, pl.ds(col + 32, 16)]
            dnxt = val_v[row, pl.ds(col + 48, 16)]
            dc = scur - dcur
            dn = snxt - dnxt
            diff = dc + dtg * (dn - dc)
            return acc + diff * diff

        acc = lax.fori_loop(0, DIM, cd, jnp.zeros((16,), jnp.float32))
        out_v[pl.ds(g * 16, 16)] = bias_v[...] - acc

    build_issue(0)
    build_issue(1)

    def step(i, carry):
        build_issue(i + 2)
        drain_compute(i)
        return carry

    lax.fori_loop(0, NG - 2, step, 0)
    drain_compute(NG - 2)
    drain_compute(NG - 1)
    pltpu.sync_copy(out_v, out.at[pl.ds(base, BPW)])


def kernel(z, bias, src, dst, t):
    flat = jnp.transpose(z, (2, 1, 0)).reshape(-1)
    bias16 = jnp.broadcast_to(jnp.asarray(bias, jnp.float32), (16,))
    src2 = src.astype(jnp.int32).reshape(NW, BPW)
    dst2 = dst.astype(jnp.int32).reshape(NW, BPW)
    t2 = t.reshape(NW, BPW)
    fn = pl.kernel(
        _body,
        out_type=jax.ShapeDtypeStruct((BATCH,), jnp.float32),
        mesh=plsc.VectorSubcoreMesh(core_axis_name="c", subcore_axis_name="s"),
        compiler_params=pltpu.CompilerParams(use_tc_tiling_on_sc=False,
                                             needs_layout_passes=False,
                                             skip_device_barrier=True),
        scratch_types=[
            pltpu.VMEM((BPW,), jnp.int32),
            pltpu.VMEM((BPW,), jnp.int32),
            pltpu.VMEM((BPW,), jnp.float32),
            pltpu.VMEM((BPW,), jnp.int32),
            pltpu.VMEM((BPW,), jnp.float32),
            pltpu.VMEM((NROW, 128), jnp.int32),
            pltpu.VMEM((NROW, 128), jnp.float32),
            pltpu.VMEM((BPW,), jnp.float32),
            pltpu.VMEM((16,), jnp.float32),
            pltpu.SemaphoreType.DMA,
        ],
    )
    return fn(flat, bias16, src2, dst2, t2)
